# Initial kernel scaffold; baseline (speedup 1.0000x reference)
import jax
import jax.numpy as jnp
from jax.experimental import pallas as pl


def kernel(z, edge_index, pos_edge_index):
    raise NotImplementedError("write your pallas kernel here")


# speedup vs baseline: 30.9969x; 30.9969x over previous
import jax
import jax.numpy as jnp
from jax import lax
from jax.experimental import pallas as pl
from jax.experimental.pallas import tpu as pltpu
from jax.experimental.pallas import tpu_sc as plsc

N_NODES = 10000
N_EDGES = 320000

NUM_CORES = 2
NUM_SUBCORES = 16
LANES = 16
NUM_TILES = NUM_CORES * NUM_SUBCORES

BIN_COLS = 128
BIN_ROWS = 80
NUM_BINS = BIN_ROWS * BIN_COLS

SRC_PER_TILE = N_EDGES // NUM_SUBCORES
HEAD_PER_TILE = N_EDGES // NUM_TILES


def _degree_kernel_body(
    src_hbm,
    head_hbm,
    out_hbm,
    src_v,
    head_v,
    hist_v,
    hist2d_v,
    deg_v,
    deg2d_v,
    rowidx_v,
    out_v,
    acc_sh,
):
    c = lax.axis_index("c")
    s = lax.axis_index("s")
    wid = c * NUM_SUBCORES + s

    pltpu.sync_copy(src_hbm.at[pl.ds(s * SRC_PER_TILE, SRC_PER_TILE)], src_v)
    pltpu.sync_copy(head_hbm.at[pl.ds(wid * HEAD_PER_TILE, HEAD_PER_TILE)], head_v)

    def init_rows(j, _):
        rowidx_v[pl.ds(j * LANES, LANES)] = lax.iota(jnp.int32, LANES) + j * LANES
        return _

    lax.fori_loop(0, BIN_ROWS // LANES, init_rows, None)

    zeros = jnp.zeros((LANES,), jnp.float32)

    def zero_hist(j, _):
        hist_v[pl.ds(j * LANES, LANES)] = zeros
        return _

    lax.fori_loop(0, NUM_BINS // LANES, zero_hist, None)

    ones = jnp.ones((LANES,), jnp.float32)

    def hist_step(i, _):
        v = src_v[pl.ds(i * LANES, LANES)]
        plsc.addupdate_scatter(hist_v, [v], ones)
        return _

    lax.fori_loop(0, SRC_PER_TILE // LANES, hist_step, None)

    def pack_step(j, _):
        hist2d_v[j >> 3, pl.ds((j & 7) * LANES, LANES)] = hist_v[pl.ds(j * LANES, LANES)]
        return _

    lax.fori_loop(0, NUM_BINS // LANES, pack_step, None)

    @pl.when(s == 0)
    def _():
        pltpu.sync_copy(hist2d_v, acc_sh)

    plsc.subcore_barrier()

    @pl.when(s != 0)
    def _():
        pltpu.sync_copy(hist2d_v, acc_sh.at[rowidx_v], add=True)

    plsc.subcore_barrier()

    pltpu.sync_copy(acc_sh, deg2d_v)

    def unpack_step(j, _):
        deg_v[pl.ds(j * LANES, LANES)] = deg2d_v[j >> 3, pl.ds((j & 7) * LANES, LANES)]
        return _

    lax.fori_loop(0, NUM_BINS // LANES, unpack_step, None)

    def gather_step(i, _):
        h = head_v[pl.ds(i * LANES, LANES)]
        out_v[pl.ds(i * LANES, LANES)] = plsc.load_gather(deg_v, [h])
        return _

    lax.fori_loop(0, HEAD_PER_TILE // LANES, gather_step, None)

    pltpu.sync_copy(out_v, out_hbm.at[pl.ds(wid * HEAD_PER_TILE, HEAD_PER_TILE)])


@jax.jit
def _degree_call(src, head):
    return pl.kernel(
        _degree_kernel_body,
        out_type=jax.ShapeDtypeStruct((N_EDGES,), jnp.float32),
        compiler_params=pltpu.CompilerParams(needs_layout_passes=False),
        mesh=plsc.VectorSubcoreMesh(
            core_axis_name="c",
            subcore_axis_name="s",
            num_cores=NUM_CORES,
            num_subcores=NUM_SUBCORES,
        ),
        scratch_types=[
            pltpu.VMEM((SRC_PER_TILE,), jnp.int32),
            pltpu.VMEM((HEAD_PER_TILE,), jnp.int32),
            pltpu.VMEM((NUM_BINS,), jnp.float32),
            pltpu.VMEM((BIN_ROWS, BIN_COLS), jnp.float32),
            pltpu.VMEM((NUM_BINS,), jnp.float32),
            pltpu.VMEM((BIN_ROWS, BIN_COLS), jnp.float32),
            pltpu.VMEM((BIN_ROWS,), jnp.int32),
            pltpu.VMEM((HEAD_PER_TILE,), jnp.float32),
            pltpu.VMEM_SHARED((BIN_ROWS, BIN_COLS), jnp.float32),
        ],
    )(src, head)


def kernel(z, edge_index, pos_edge_index):
    del z
    head = edge_index[0, :].astype(jnp.int32)
    src = pos_edge_index[0, :].astype(jnp.int32)
    return _degree_call(src, head)


# speedup vs baseline: 33.7664x; 1.0893x over previous
import jax
import jax.numpy as jnp
from jax import lax
from jax.experimental import pallas as pl
from jax.experimental.pallas import tpu as pltpu
from jax.experimental.pallas import tpu_sc as plsc

N_NODES = 10000
N_EDGES = 320000

NUM_SUBCORES = 16
LANES = 16

BIN_COLS = 128
BIN_ROWS = 80
NUM_BINS = BIN_ROWS * BIN_COLS

E_PER_TILE = N_EDGES // NUM_SUBCORES


def _degree_kernel_body(
    src_hbm,
    head_hbm,
    out_hbm,
    src_v,
    head_v,
    hist_a,
    hist_b,
    stage2d_v,
    deg_v,
    rowidx_v,
    out_v,
    acc_sh,
    src_sem,
    head_sem,
):
    s = lax.axis_index("s")

    src_copy = pltpu.async_copy(
        src_hbm.at[pl.ds(s * E_PER_TILE, E_PER_TILE)], src_v, src_sem
    )
    head_copy = pltpu.async_copy(
        head_hbm.at[pl.ds(s * E_PER_TILE, E_PER_TILE)], head_v, head_sem
    )

    def init_rows(j, _):
        rowidx_v[pl.ds(j * LANES, LANES)] = lax.iota(jnp.int32, LANES) + j * LANES
        return _

    lax.fori_loop(0, BIN_ROWS // LANES, init_rows, None)

    zeros = jnp.zeros((LANES,), jnp.float32)

    def zero_hist(j, _):
        hist_a[pl.ds(j * LANES, LANES)] = zeros
        hist_b[pl.ds(j * LANES, LANES)] = zeros
        return _

    lax.fori_loop(0, NUM_BINS // LANES, zero_hist, None)

    src_copy.wait()

    ones = jnp.ones((LANES,), jnp.float32)

    def hist_step(i, _):
        v0 = src_v[pl.ds(i * (2 * LANES), LANES)]
        v1 = src_v[pl.ds(i * (2 * LANES) + LANES, LANES)]
        plsc.addupdate_scatter(hist_a, [v0], ones)
        plsc.addupdate_scatter(hist_b, [v1], ones)
        return _

    lax.fori_loop(0, E_PER_TILE // (2 * LANES), hist_step, None)

    def pack_step(j, _):
        stage2d_v[j >> 3, pl.ds((j & 7) * LANES, LANES)] = (
            hist_a[pl.ds(j * LANES, LANES)] + hist_b[pl.ds(j * LANES, LANES)]
        )
        return _

    lax.fori_loop(0, NUM_BINS // LANES, pack_step, None)

    @pl.when(s == 0)
    def _():
        pltpu.sync_copy(stage2d_v, acc_sh)

    plsc.subcore_barrier()

    @pl.when(s != 0)
    def _():
        pltpu.sync_copy(stage2d_v, acc_sh.at[rowidx_v], add=True)

    plsc.subcore_barrier()

    pltpu.sync_copy(acc_sh, stage2d_v)

    def unpack_step(j, _):
        deg_v[pl.ds(j * LANES, LANES)] = stage2d_v[j >> 3, pl.ds((j & 7) * LANES, LANES)]
        return _

    lax.fori_loop(0, NUM_BINS // LANES, unpack_step, None)

    head_copy.wait()

    def gather_step(i, _):
        h0 = head_v[pl.ds(i * (2 * LANES), LANES)]
        h1 = head_v[pl.ds(i * (2 * LANES) + LANES, LANES)]
        out_v[pl.ds(i * (2 * LANES), LANES)] = plsc.load_gather(deg_v, [h0])
        out_v[pl.ds(i * (2 * LANES) + LANES, LANES)] = plsc.load_gather(deg_v, [h1])
        return _

    lax.fori_loop(0, E_PER_TILE // (2 * LANES), gather_step, None)

    pltpu.sync_copy(out_v, out_hbm.at[pl.ds(s * E_PER_TILE, E_PER_TILE)])


@jax.jit
def _degree_call(src, head):
    return pl.kernel(
        _degree_kernel_body,
        out_type=jax.ShapeDtypeStruct((N_EDGES,), jnp.float32),
        compiler_params=pltpu.CompilerParams(needs_layout_passes=False),
        mesh=plsc.VectorSubcoreMesh(
            core_axis_name="c",
            subcore_axis_name="s",
            num_cores=1,
            num_subcores=NUM_SUBCORES,
        ),
        scratch_types=[
            pltpu.VMEM((E_PER_TILE,), jnp.int32),
            pltpu.VMEM((E_PER_TILE,), jnp.int32),
            pltpu.VMEM((NUM_BINS,), jnp.float32),
            pltpu.VMEM((NUM_BINS,), jnp.float32),
            pltpu.VMEM((BIN_ROWS, BIN_COLS), jnp.float32),
            pltpu.VMEM((NUM_BINS,), jnp.float32),
            pltpu.VMEM((BIN_ROWS,), jnp.int32),
            pltpu.VMEM((E_PER_TILE,), jnp.float32),
            pltpu.VMEM_SHARED((BIN_ROWS, BIN_COLS), jnp.float32),
            pltpu.SemaphoreType.DMA,
            pltpu.SemaphoreType.DMA,
        ],
    )(src, head)


def kernel(z, edge_index, pos_edge_index):
    del z
    head = edge_index[0, :].astype(jnp.int32)
    src = pos_edge_index[0, :].astype(jnp.int32)
    return _degree_call(src, head)
